# baseline (speedup 1.0000x reference)
import functools

import jax
import jax.numpy as jnp
from jax import lax
from jax.experimental import pallas as pl
from jax.experimental.pallas import tpu as pltpu
from jax.experimental.pallas import tpu_sc as plsc

_NC, _NS = 2, 16
_NW = _NC * _NS
_ROWS = 16
_LANES = 16


def kernel(x, pe_weight):
    B, S, D = x.shape
    SD = S * D
    ch = _ROWS * D
    rows_per_w = S // _NW
    chunks = rows_per_w // _ROWS

    xf = x.reshape(B * SD)
    pef = pe_weight.reshape(SD)

    mesh = plsc.VectorSubcoreMesh(
        core_axis_name="c", subcore_axis_name="s",
        num_cores=_NC, num_subcores=_NS)

    @functools.partial(
        pl.kernel,
        out_type=jax.ShapeDtypeStruct((B * SD,), jnp.float32),
        mesh=mesh,
        scratch_types=[
            pltpu.VMEM((ch,), jnp.float32),
            pltpu.VMEM((ch,), jnp.float32),
        ],
    )
    def run(x_hbm, pe_hbm, out_hbm, pebuf, xbuf):
        w = lax.axis_index("s") * _NC + lax.axis_index("c")
        seq_base = w * rows_per_w * D

        def chunk_body(c, _):
            pe_off = seq_base + c * ch
            pltpu.sync_copy(pe_hbm.at[pl.ds(pe_off, ch)], pebuf)

            def batch_body(b, carry):
                x_off = b * SD + pe_off
                pltpu.sync_copy(x_hbm.at[pl.ds(x_off, ch)], xbuf)

                def add_body(i, acc):
                    plsc.addupdate(xbuf.at[pl.ds(i * _LANES, _LANES)],
                                   pebuf[pl.ds(i * _LANES, _LANES)])
                    return acc

                lax.fori_loop(0, ch // _LANES, add_body, 0, unroll=8)
                pltpu.sync_copy(xbuf, out_hbm.at[pl.ds(x_off, ch)])
                return carry

            lax.fori_loop(0, B, batch_body, 0)
            return _

        lax.fori_loop(0, chunks, chunk_body, 0)

    return run(xf, pef).reshape(B, S, D)


# speedup vs baseline: 1.3253x; 1.3253x over previous
import functools

import jax
import jax.numpy as jnp
from jax import lax
from jax.experimental import pallas as pl
from jax.experimental.pallas import tpu as pltpu
from jax.experimental.pallas import tpu_sc as plsc

_NC, _NS = 2, 16
_NW = _NC * _NS
_ROWS = 16
_L = 16


def kernel(x, pe_weight):
    B, S, D = x.shape
    SD = S * D
    ch = _ROWS * D
    rows_per_w = S // _NW
    chunks = rows_per_w // _ROWS
    T = chunks // 2

    xf = x.reshape(B * SD)
    pef = pe_weight.reshape(SD)

    mesh = plsc.VectorSubcoreMesh(
        core_axis_name="c", subcore_axis_name="s",
        num_cores=_NC, num_subcores=_NS)

    @functools.partial(
        pl.kernel,
        out_type=jax.ShapeDtypeStruct((B * SD,), jnp.float32),
        mesh=mesh,
        scratch_types=[
            [pltpu.VMEM((ch,), jnp.float32) for _ in range(2)],
            [pltpu.VMEM((ch,), jnp.float32) for _ in range(4)],
            [pltpu.SemaphoreType.DMA for _ in range(2)],
            [pltpu.SemaphoreType.DMA for _ in range(4)],
            [pltpu.SemaphoreType.DMA for _ in range(4)],
        ],
    )
    def run(x_hbm, pe_hbm, out_hbm, pebufs, xbufs, pe_sems, xin_sems, out_sems):
        w = lax.axis_index("s") * _NC + lax.axis_index("c")
        seq_base = w * rows_per_w * D

        def pe_off(c):
            return seq_base + c * ch

        def x_off(c, b):
            return b * SD + pe_off(c)

        def issue_pe(c, k):
            pltpu.async_copy(pe_hbm.at[pl.ds(pe_off(c), ch)], pebufs[k],
                             pe_sems[k])

        def wait_pe(c, k):
            pltpu.make_async_copy(pe_hbm.at[pl.ds(pe_off(c), ch)], pebufs[k],
                                  pe_sems[k]).wait()

        def issue_xin(c, b, k):
            pltpu.async_copy(x_hbm.at[pl.ds(x_off(c, b), ch)], xbufs[k],
                             xin_sems[k])

        def wait_xin(c, b, k):
            pltpu.make_async_copy(x_hbm.at[pl.ds(x_off(c, b), ch)], xbufs[k],
                                  xin_sems[k]).wait()

        def issue_out(c, b, k):
            pltpu.async_copy(xbufs[k], out_hbm.at[pl.ds(x_off(c, b), ch)],
                             out_sems[k])

        def wait_out(c, b, k):
            pltpu.make_async_copy(xbufs[k], out_hbm.at[pl.ds(x_off(c, b), ch)],
                                  out_sems[k]).wait()

        def add_stage(k, pk):
            def add_body(i, acc):
                plsc.addupdate(xbufs[k].at[pl.ds(i * _L, _L)],
                               pebufs[pk][pl.ds(i * _L, _L)])
                return acc
            lax.fori_loop(0, ch // _L, add_body, 0, unroll=8)

        issue_pe(0, 0)
        for b in range(4):
            issue_xin(0, b, b)

        def body(t, carry):
            for cc in range(2):
                c = 2 * t + cc
                for b in range(4):
                    if b == 0:
                        wait_pe(c, cc)
                        if cc == 0:
                            issue_pe(c + 1, 1)
                        else:
                            @pl.when(t < T - 1)
                            def _():
                                issue_pe(c + 1, 0)
                    wait_xin(c, b, b)
                    add_stage(b, cc)
                    issue_out(c, b, b)
                    kb = (b + 3) % 4
                    if cc == 0 and b == 0:
                        @pl.when(t >= 1)
                        def _():
                            wait_out(c - 1, 3, kb)
                            issue_xin(c, 3, kb)
                    elif cc == 1 and b >= 1:
                        @pl.when(t < T - 1)
                        def _():
                            wait_out(c, b - 1, kb)
                            issue_xin(c + 1, b - 1, kb)
                    elif b == 0:
                        wait_out(c, 3, kb)
                        issue_xin(c + 1, 3, kb)
                    else:
                        wait_out(c, b - 1, kb)
                        issue_xin(c + 1, b - 1, kb)
            return carry

        lax.fori_loop(0, T, body, 0)

        for b in range(4):
            wait_out(chunks - 1, b, b)

    return run(xf, pef).reshape(B, S, D)


# speedup vs baseline: 1.7098x; 1.2901x over previous
import functools

import jax
import jax.numpy as jnp
from jax import lax
from jax.experimental import pallas as pl
from jax.experimental.pallas import tpu as pltpu
from jax.experimental.pallas import tpu_sc as plsc

_NC, _NS = 2, 16
_NW = _NC * _NS
_ROWS = 16
_L = 16


def kernel(x, pe_weight):
    B, S, D = x.shape
    rows_per_w = S // _NW
    chunks = rows_per_w // _ROWS
    T = chunks // 2

    xf = x.reshape(B * S, D)

    mesh = plsc.VectorSubcoreMesh(
        core_axis_name="c", subcore_axis_name="s",
        num_cores=_NC, num_subcores=_NS)

    @functools.partial(
        pl.kernel,
        out_type=jax.ShapeDtypeStruct((B * S, D), jnp.float32),
        mesh=mesh,
        compiler_params=pltpu.CompilerParams(use_tc_tiling_on_sc=True),
        scratch_types=[
            [pltpu.VMEM((_ROWS, D), jnp.float32) for _ in range(2)],
            [pltpu.VMEM((_ROWS, D), jnp.float32) for _ in range(4)],
            [pltpu.SemaphoreType.DMA for _ in range(2)],
            [pltpu.SemaphoreType.DMA for _ in range(4)],
            [pltpu.SemaphoreType.DMA for _ in range(4)],
        ],
    )
    def run(x_hbm, pe_hbm, out_hbm, pebufs, xbufs, pe_sems, xin_sems, out_sems):
        w = lax.axis_index("s") * _NC + lax.axis_index("c")
        seq_base = w * rows_per_w

        def pe_row(c):
            return seq_base + c * _ROWS

        def x_row(c, b):
            return b * S + pe_row(c)

        def issue_pe(c, k):
            pltpu.async_copy(pe_hbm.at[pl.ds(pe_row(c), _ROWS), :], pebufs[k],
                             pe_sems[k])

        def wait_pe(c, k):
            pltpu.make_async_copy(pe_hbm.at[pl.ds(pe_row(c), _ROWS), :],
                                  pebufs[k], pe_sems[k]).wait()

        def issue_xin(c, b, k):
            pltpu.async_copy(x_hbm.at[pl.ds(x_row(c, b), _ROWS), :], xbufs[k],
                             xin_sems[k])

        def wait_xin(c, b, k):
            pltpu.make_async_copy(x_hbm.at[pl.ds(x_row(c, b), _ROWS), :],
                                  xbufs[k], xin_sems[k]).wait()

        def issue_out(c, b, k):
            pltpu.async_copy(xbufs[k], out_hbm.at[pl.ds(x_row(c, b), _ROWS), :],
                             out_sems[k])

        def wait_out(c, b, k):
            pltpu.make_async_copy(xbufs[k],
                                  out_hbm.at[pl.ds(x_row(c, b), _ROWS), :],
                                  out_sems[k]).wait()

        def add_stage(k, pk):
            def row_body(r, carry):
                def add_body(j, acc):
                    plsc.addupdate(xbufs[k].at[r, pl.ds(j * _L, _L)],
                                   pebufs[pk][r, pl.ds(j * _L, _L)])
                    return acc
                lax.fori_loop(0, D // _L, add_body, 0, unroll=8)
                return carry
            lax.fori_loop(0, _ROWS, row_body, 0)

        issue_pe(0, 0)
        for b in range(4):
            issue_xin(0, b, b)

        def body(t, carry):
            for cc in range(2):
                c = 2 * t + cc
                for b in range(4):
                    if b == 0:
                        wait_pe(c, cc)
                        if cc == 0:
                            issue_pe(c + 1, 1)
                        else:
                            @pl.when(t < T - 1)
                            def _():
                                issue_pe(c + 1, 0)
                    wait_xin(c, b, b)
                    add_stage(b, cc)
                    issue_out(c, b, b)
                    kb = (b + 3) % 4
                    if cc == 0 and b == 0:
                        @pl.when(t >= 1)
                        def _():
                            wait_out(c - 1, 3, kb)
                            issue_xin(c, 3, kb)
                    elif cc == 1 and b >= 1:
                        @pl.when(t < T - 1)
                        def _():
                            wait_out(c, b - 1, kb)
                            issue_xin(c + 1, b - 1, kb)
                    elif b == 0:
                        wait_out(c, 3, kb)
                        issue_xin(c + 1, 3, kb)
                    else:
                        wait_out(c, b - 1, kb)
                        issue_xin(c + 1, b - 1, kb)
            return carry

        lax.fori_loop(0, T, body, 0)

        for b in range(4):
            wait_out(chunks - 1, b, b)

    return run(xf, pe_weight).reshape(B, S, D)
